# baseline (speedup 1.0000x reference)
import functools

import jax
import jax.numpy as jnp
from jax.experimental import pallas as pl
from jax.experimental.pallas import tpu as pltpu

_N = 10000
_F = 128
_BM = 400
_NSPLIT = 2
_H = _BM // _NSPLIT


def _gnn_kernel(active_ref, features_ref, w_ref, *rest):
    adj_refs = rest[:_NSPLIT]
    out_ref = rest[_NSPLIT]
    support_ref = rest[_NSPLIT + 1]
    i = pl.program_id(0)

    @pl.when(i == 0)
    def _():
        s = jnp.dot(features_ref[...], w_ref[...],
                    preferred_element_type=jnp.float32)
        support_ref[...] = jnp.where(active_ref[0] != 0, jnp.tanh(s), s)

    for j in range(_NSPLIT):
        out_ref[pl.ds(j * _H, _H), :] = jnp.dot(
            adj_refs[j][...], support_ref[...],
            preferred_element_type=jnp.float32)


def kernel(features, adj, W, active):
    active_arr = jnp.asarray(active, jnp.int32).reshape((1,))
    adj_specs = [
        pl.BlockSpec((_H, _N), functools.partial(
            lambda j, i, a: (i * _NSPLIT + j, 0), j))
        for j in range(_NSPLIT)
    ]
    return pl.pallas_call(
        _gnn_kernel,
        grid_spec=pltpu.PrefetchScalarGridSpec(
            num_scalar_prefetch=1,
            grid=(_N // _BM,),
            in_specs=[
                pl.BlockSpec((_N, _F), lambda i, a: (0, 0)),
                pl.BlockSpec((_F, _F), lambda i, a: (0, 0)),
                *adj_specs,
            ],
            out_specs=pl.BlockSpec((_BM, _F), lambda i, a: (i, 0)),
            scratch_shapes=[pltpu.VMEM((_N, _F), jnp.float32)],
        ),
        out_shape=jax.ShapeDtypeStruct((_N, _F), jnp.float32),
        compiler_params=pltpu.CompilerParams(
            dimension_semantics=("arbitrary",),
        ),
    )(active_arr, features, W, *([adj] * _NSPLIT))
